# baseline (speedup 1.0000x reference)
import functools

import jax
import jax.numpy as jnp
from jax import lax
from jax.experimental import pallas as pl
from jax.experimental.pallas import tpu as pltpu
from jax.experimental.pallas import tpu_sc as plsc

_B, _T, _D, _S, _NT = 16, 2048, 128, 512, 8192
_L = 16
_DR = _D * _L
_CH = 16
_NCH = _T // _CH
_UNROLL = 8


def _sc_forward_body(obs_hbm, packed_hbm, prob_hbm, leaky_hbm, final_hbm,
                     len_hbm, scales_out, fp_out,
                     obs2, packed_v, prob_v, leaky_v, final_v, alpha_v, anew_v,
                     scales_v, fp_v, len_v, sem0, sem1):
    g = lax.axis_index("c")
    b = lax.axis_index("s")
    chain = g * _B + b

    pltpu.sync_copy(packed_hbm.at[g], packed_v)
    pltpu.sync_copy(prob_hbm.at[g], prob_v)
    pltpu.sync_copy(leaky_hbm.at[g], leaky_v)
    pltpu.sync_copy(final_hbm.at[g], final_v)
    pltpu.sync_copy(len_hbm, len_v)

    zero16 = jnp.zeros((_L,), jnp.float32)
    iota16 = lax.iota(jnp.int32, _L)
    lane0 = iota16 == 0

    for m in range(_S // _L):
        alpha_v[pl.ds(m * _L, _L)] = zero16
        anew_v[pl.ds(m * _L, _L)] = zero16
    alpha_v[pl.ds(0, _L)] = jnp.where(lane0, 1.0, 0.0).astype(jnp.float32)

    gdn = lax.GatherDimensionNumbers(
        offset_dims=(), collapsed_slice_dims=(0,), start_index_map=(0,))

    def xlane_sum(v):
        for sh in (8, 4, 2, 1):
            perm = lax.gather(v, (iota16 ^ sh)[:, None], gdn, (1,),
                              mode=lax.GatherScatterMode.PROMISE_IN_BOUNDS)
            v = v + perm
        return v

    acc = zero16
    for m in range(_S // _L):
        acc = acc + leaky_v[pl.ds(m * _L, _L)]
    sum_leaky = xlane_sum(acc)

    leaky_coef = jnp.where(g == 0, 1e-5, 1e-20).astype(jnp.float32)

    lv = len_v[pl.ds(0, _L)]
    lenb = lax.gather(lv, jnp.full((_L, 1), b, jnp.int32), gdn, (1,),
                      mode=lax.GatherScatterMode.PROMISE_IN_BOUNDS)

    pltpu.async_copy(obs_hbm.at[b, pl.ds(0, _CH * _DR)],
                     obs2.at[pl.ds(0, _CH * _DR)], sem0)

    def do_chunk(c, slot):
        def step(i, _):
            base = jnp.full((_L,), (slot * _CH + i) * _DR, jnp.int32) + iota16

            @plsc.parallel_loop(0, _NT, _L, unroll=_UNROLL)
            def grp(off):
                pk = packed_v[pl.ds(off, _L)]
                fr = pk & 511
                to = lax.shift_right_logical(pk, 9) & 511
                pdf = lax.shift_right_logical(pk, 18)
                ag = plsc.load_gather(alpha_v, [fr])
                og = plsc.load_gather(obs2, [base + (pdf << 4)])
                pr = prob_v[pl.ds(off, _L)]
                plsc.addupdate_scatter(anew_v, [to], ag * pr * og)

            tvec = jnp.full((_L,), c * _CH + i, jnp.int32)

            accs = [zero16] * 4
            for m in range(_S // _L):
                accs[m % 4] = accs[m % 4] + anew_v[pl.ds(m * _L, _L)]
            tot = xlane_sum((accs[0] + accs[1]) + (accs[2] + accs[3]))
            scale = tot + leaky_coef * tot * sum_leaky + 1e-30
            inv = 1.0 / scale
            active = tvec < lenb
            scale_vec = jnp.where(active, scale, 1.0)
            plsc.store_scatter(scales_v, [tvec], scale_vec, mask=lane0)
            ltotv = leaky_coef * tot
            invv = inv
            for m in range(_S // _L):
                ds_ = pl.ds(m * _L, _L)
                nrm = (anew_v[ds_] + ltotv * leaky_v[ds_]) * invv
                alpha_v[ds_] = jnp.where(active, nrm, alpha_v[ds_])
                anew_v[ds_] = zero16
            return 0

        lax.fori_loop(0, _CH, step, 0)

    def pair(c2, _):
        pltpu.make_async_copy(
            obs_hbm.at[b, pl.ds(0, _CH * _DR)],
            obs2.at[pl.ds(0, _CH * _DR)], sem0).wait()
        pltpu.async_copy(
            obs_hbm.at[b, pl.ds((2 * c2 + 1) * (_CH * _DR), _CH * _DR)],
            obs2.at[pl.ds(_CH * _DR, _CH * _DR)], sem1)
        do_chunk(2 * c2, 0)

        @pl.when(c2 < _NCH // 2 - 1)
        def _():
            pltpu.async_copy(
                obs_hbm.at[b, pl.ds((2 * c2 + 2) * (_CH * _DR), _CH * _DR)],
                obs2.at[pl.ds(0, _CH * _DR)], sem0)

        pltpu.make_async_copy(
            obs_hbm.at[b, pl.ds(0, _CH * _DR)],
            obs2.at[pl.ds(_CH * _DR, _CH * _DR)], sem1).wait()
        do_chunk(2 * c2 + 1, 1)
        return 0

    lax.fori_loop(0, _NCH // 2, pair, 0)

    accf = zero16
    for m in range(_S // _L):
        accf = accf + alpha_v[pl.ds(m * _L, _L)] * final_v[pl.ds(m * _L, _L)]
    fp_v[pl.ds(0, _L)] = accf

    pltpu.sync_copy(scales_v, scales_out.at[chain])
    pltpu.sync_copy(fp_v, fp_out.at[chain])


_sc_forward = functools.partial(
    pl.kernel,
    mesh=plsc.VectorSubcoreMesh(core_axis_name="c", subcore_axis_name="s"),
    compiler_params=pltpu.CompilerParams(needs_layout_passes=False),
    out_type=[
        jax.ShapeDtypeStruct((2 * _B, _T), jnp.float32),
        jax.ShapeDtypeStruct((2 * _B, _L), jnp.float32),
    ],
    scratch_types=[
        pltpu.VMEM((2 * _CH * _DR,), jnp.float32),
        pltpu.VMEM((_NT,), jnp.int32),
        pltpu.VMEM((_NT,), jnp.float32),
        pltpu.VMEM((_S,), jnp.float32),
        pltpu.VMEM((_S,), jnp.float32),
        pltpu.VMEM((_S,), jnp.float32),
        pltpu.VMEM((_S,), jnp.float32),
        pltpu.VMEM((_T,), jnp.float32),
        pltpu.VMEM((_L,), jnp.float32),
        pltpu.VMEM((_B,), jnp.int32),
        pltpu.SemaphoreType.DMA,
        pltpu.SemaphoreType.DMA,
    ],
)(_sc_forward_body)


def _exp_body(x_ref, o_ref):
    e = jnp.exp(jnp.clip(x_ref[...], -30.0, 30.0))
    o_ref[...] = jnp.repeat(e, _L, axis=2)


def _reduce_body(scales_ref, fp_ref, len_ref, o_ref):
    per = jnp.sum(jnp.log(scales_ref[...]), axis=1, keepdims=True)
    fin = jnp.sum(fp_ref[...], axis=1, keepdims=True)
    per = per + jnp.log(fin + 1e-30)
    den = jnp.sum(per[:_B])
    num = jnp.sum(per[_B:])
    tl = jnp.sum(len_ref[...])
    o_ref[...] = (-(num - den) / tl).reshape(1, 1)


def _prep_graph(t_from, t_to, t_pdf, t_prob):
    packed = (t_from | (t_to << 9) | (t_pdf << 18)).astype(jnp.int32)
    order = jnp.argsort((t_to % _L) * _S + t_to)
    order = order.reshape(_L, _NT // _L).T.reshape(-1)
    return packed[order], t_prob[order]


def kernel(x, x_lengths, den_from, den_to, den_pdf, den_prob, den_leaky,
           den_final, num_from, num_to, num_pdf, num_prob, num_leaky,
           num_final):
    obs = pl.pallas_call(
        _exp_body,
        out_shape=jax.ShapeDtypeStruct((_B, _T, _DR), jnp.float32),
        grid=(_B, 8),
        in_specs=[pl.BlockSpec((1, _T // 8, _D), lambda i, j: (i, j, 0))],
        out_specs=pl.BlockSpec((1, _T // 8, _DR), lambda i, j: (i, j, 0)),
    )(x).reshape(_B, _T * _DR)

    dp, dpr = _prep_graph(den_from, den_to, den_pdf, den_prob)
    np_, npr = _prep_graph(num_from, num_to, num_pdf, num_prob)
    packed = jnp.stack([dp, np_])
    prob = jnp.stack([dpr, npr])
    leaky = jnp.stack([den_leaky, num_leaky])
    final = jnp.stack([den_final, num_final])

    scales, fp = _sc_forward(obs, packed, prob, leaky, final,
                             x_lengths.astype(jnp.int32))

    out = pl.pallas_call(
        _reduce_body,
        out_shape=jax.ShapeDtypeStruct((1, 1), jnp.float32),
    )(scales, fp, x_lengths.astype(jnp.float32).reshape(1, _B))
    return out.reshape(())


# speedup vs baseline: 1.2676x; 1.2676x over previous
import functools

import jax
import jax.numpy as jnp
from jax import lax
from jax.experimental import pallas as pl
from jax.experimental.pallas import tpu as pltpu
from jax.experimental.pallas import tpu_sc as plsc

_B, _T, _D, _S, _NT = 16, 2048, 128, 512, 8192
_L = 16
_CH = 64
_NCH = _T // _CH
_UNROLL = 8


def _sc_forward_body(obs_hbm, packed_hbm, prob_hbm, leaky_hbm, final_hbm,
                     len_hbm, scales_out, fp_out,
                     obs2, packed_v, prob_v, leaky_v, final_v, alpha_v, anew_v,
                     scales_v, fp_v, len_v, sem0, sem1):
    g = lax.axis_index("c")
    b = lax.axis_index("s")
    chain = g * _B + b

    pltpu.sync_copy(packed_hbm.at[g], packed_v)
    pltpu.sync_copy(prob_hbm.at[g], prob_v)
    pltpu.sync_copy(leaky_hbm.at[g], leaky_v)
    pltpu.sync_copy(final_hbm.at[g], final_v)
    pltpu.sync_copy(len_hbm, len_v)

    zero16 = jnp.zeros((_L,), jnp.float32)
    iota16 = lax.iota(jnp.int32, _L)
    lane0 = iota16 == 0

    for m in range(_S // _L):
        alpha_v[pl.ds(m * _L, _L)] = zero16
        anew_v[pl.ds(m * _L, _L)] = zero16
    alpha_v[pl.ds(0, _L)] = jnp.where(lane0, 1.0, 0.0).astype(jnp.float32)

    gdn = lax.GatherDimensionNumbers(
        offset_dims=(), collapsed_slice_dims=(0,), start_index_map=(0,))

    def xlane_sum(v):
        for sh in (8, 4, 2, 1):
            perm = lax.gather(v, (iota16 ^ sh)[:, None], gdn, (1,),
                              mode=lax.GatherScatterMode.PROMISE_IN_BOUNDS)
            v = v + perm
        return v

    acc = zero16
    for m in range(_S // _L):
        acc = acc + leaky_v[pl.ds(m * _L, _L)]
    sum_leaky = xlane_sum(acc)

    leaky_coef = jnp.where(g == 0, 1e-5, 1e-20).astype(jnp.float32)

    lv = len_v[pl.ds(0, _L)]
    lenb = lax.gather(lv, jnp.full((_L, 1), b, jnp.int32), gdn, (1,),
                      mode=lax.GatherScatterMode.PROMISE_IN_BOUNDS)

    pltpu.async_copy(obs_hbm.at[b, pl.ds(0, _CH * _D)],
                     obs2.at[pl.ds(0, _CH * _D)], sem0)

    def do_chunk(c, slot):
        def step(i, _):
            base = jnp.full((_L,), (slot * _CH + i) * _D, jnp.int32)

            @plsc.parallel_loop(0, _NT, _L, unroll=_UNROLL)
            def grp(off):
                pk = packed_v[pl.ds(off, _L)]
                fr = pk & 511
                to = lax.shift_right_logical(pk, 9) & 511
                pdf = lax.shift_right_logical(pk, 18)
                ag = plsc.load_gather(alpha_v, [fr])
                og = plsc.load_gather(obs2, [base + pdf])
                pr = prob_v[pl.ds(off, _L)]
                plsc.addupdate_scatter(anew_v, [to], ag * pr * og)

            tvec = jnp.full((_L,), c * _CH + i, jnp.int32)

            accs = [zero16] * 4
            for m in range(_S // _L):
                accs[m % 4] = accs[m % 4] + anew_v[pl.ds(m * _L, _L)]
            tot = xlane_sum((accs[0] + accs[1]) + (accs[2] + accs[3]))
            scale = tot + leaky_coef * tot * sum_leaky + 1e-30
            inv = 1.0 / scale
            active = tvec < lenb
            scale_vec = jnp.where(active, scale, 1.0)
            plsc.store_scatter(scales_v, [tvec], scale_vec, mask=lane0)
            ltotv = leaky_coef * tot
            invv = inv
            for m in range(_S // _L):
                ds_ = pl.ds(m * _L, _L)
                nrm = (anew_v[ds_] + ltotv * leaky_v[ds_]) * invv
                alpha_v[ds_] = jnp.where(active, nrm, alpha_v[ds_])
                anew_v[ds_] = zero16
            return 0

        lax.fori_loop(0, _CH, step, 0)

    def pair(c2, _):
        pltpu.make_async_copy(
            obs_hbm.at[b, pl.ds(0, _CH * _D)],
            obs2.at[pl.ds(0, _CH * _D)], sem0).wait()
        pltpu.async_copy(
            obs_hbm.at[b, pl.ds((2 * c2 + 1) * (_CH * _D), _CH * _D)],
            obs2.at[pl.ds(_CH * _D, _CH * _D)], sem1)
        do_chunk(2 * c2, 0)

        @pl.when(c2 < _NCH // 2 - 1)
        def _():
            pltpu.async_copy(
                obs_hbm.at[b, pl.ds((2 * c2 + 2) * (_CH * _D), _CH * _D)],
                obs2.at[pl.ds(0, _CH * _D)], sem0)

        pltpu.make_async_copy(
            obs_hbm.at[b, pl.ds(0, _CH * _D)],
            obs2.at[pl.ds(_CH * _D, _CH * _D)], sem1).wait()
        do_chunk(2 * c2 + 1, 1)
        return 0

    lax.fori_loop(0, _NCH // 2, pair, 0)

    accf = zero16
    for m in range(_S // _L):
        accf = accf + alpha_v[pl.ds(m * _L, _L)] * final_v[pl.ds(m * _L, _L)]
    fp_v[pl.ds(0, _L)] = accf

    pltpu.sync_copy(scales_v, scales_out.at[chain])
    pltpu.sync_copy(fp_v, fp_out.at[chain])


_sc_forward = functools.partial(
    pl.kernel,
    mesh=plsc.VectorSubcoreMesh(core_axis_name="c", subcore_axis_name="s"),
    compiler_params=pltpu.CompilerParams(needs_layout_passes=False),
    out_type=[
        jax.ShapeDtypeStruct((2 * _B, _T), jnp.float32),
        jax.ShapeDtypeStruct((2 * _B, _L), jnp.float32),
    ],
    scratch_types=[
        pltpu.VMEM((2 * _CH * _D,), jnp.float32),
        pltpu.VMEM((_NT,), jnp.int32),
        pltpu.VMEM((_NT,), jnp.float32),
        pltpu.VMEM((_S,), jnp.float32),
        pltpu.VMEM((_S,), jnp.float32),
        pltpu.VMEM((_S,), jnp.float32),
        pltpu.VMEM((_S,), jnp.float32),
        pltpu.VMEM((_T,), jnp.float32),
        pltpu.VMEM((_L,), jnp.float32),
        pltpu.VMEM((_B,), jnp.int32),
        pltpu.SemaphoreType.DMA,
        pltpu.SemaphoreType.DMA,
    ],
)(_sc_forward_body)


def _exp_body(x_ref, o_ref):
    o_ref[...] = jnp.exp(jnp.clip(x_ref[...], -30.0, 30.0))


def _reduce_body(scales_ref, fp_ref, len_ref, o_ref):
    per = jnp.sum(jnp.log(scales_ref[...]), axis=1, keepdims=True)
    fin = jnp.sum(fp_ref[...], axis=1, keepdims=True)
    per = per + jnp.log(fin + 1e-30)
    den = jnp.sum(per[:_B])
    num = jnp.sum(per[_B:])
    tl = jnp.sum(len_ref[...])
    o_ref[...] = (-(num - den) / tl).reshape(1, 1)


def _prep_graph(t_from, t_to, t_pdf, t_prob):
    packed = (t_from | (t_to << 9) | (t_pdf << 18)).astype(jnp.int32)
    order = jnp.argsort((t_to % _L) * _S + t_to)
    order = order.reshape(_L, _NT // _L).T.reshape(-1)
    return packed[order], t_prob[order]


def kernel(x, x_lengths, den_from, den_to, den_pdf, den_prob, den_leaky,
           den_final, num_from, num_to, num_pdf, num_prob, num_leaky,
           num_final):
    obs = pl.pallas_call(
        _exp_body,
        out_shape=jax.ShapeDtypeStruct((_B, _T, _D), jnp.float32),
        grid=(_B,),
        in_specs=[pl.BlockSpec((1, _T, _D), lambda i: (i, 0, 0))],
        out_specs=pl.BlockSpec((1, _T, _D), lambda i: (i, 0, 0)),
    )(x).reshape(_B, _T * _D)

    dp, dpr = _prep_graph(den_from, den_to, den_pdf, den_prob)
    np_, npr = _prep_graph(num_from, num_to, num_pdf, num_prob)
    packed = jnp.stack([dp, np_])
    prob = jnp.stack([dpr, npr])
    leaky = jnp.stack([den_leaky, num_leaky])
    final = jnp.stack([den_final, num_final])

    scales, fp = _sc_forward(obs, packed, prob, leaky, final,
                             x_lengths.astype(jnp.int32))

    out = pl.pallas_call(
        _reduce_body,
        out_shape=jax.ShapeDtypeStruct((1, 1), jnp.float32),
    )(scales, fp, x_lengths.astype(jnp.float32).reshape(1, _B))
    return out.reshape(())
